# baseline (speedup 1.0000x reference)
import functools

import jax
import jax.numpy as jnp
from jax import lax
from jax.experimental import pallas as pl
from jax.experimental.pallas import tpu as pltpu
from jax.experimental.pallas import tpu_sc as plsc

B, S, D = 1, 2048, 768
E, K, F = 8, 2, 2048
T = B * S
A = T * K
TM = 256
MT = A // TM + E
A_PAD = MT * TM
DH = D // 2
N_T = T // TM

_SC_WORKERS = 32
_BPW = A // _SC_WORKERS


def _pack2(y):
    yl = lax.bitcast_convert_type(y[..., :DH], jnp.int32)
    yh = lax.bitcast_convert_type(y[..., DH:], jnp.int32)
    lo = jnp.right_shift(yl + 0x8000, 16) & jnp.int32(0xFFFF)
    hi = (yh + 0x8000) & jnp.int32(-65536)
    return lo | hi


def _unpack2(u):
    lo = lax.bitcast_convert_type(jnp.left_shift(u, 16), jnp.float32)
    hi = lax.bitcast_convert_type(u & jnp.int32(-65536), jnp.float32)
    return jnp.concatenate([lo, hi], axis=-1)


def _ln_router(x, wr, lns, lnb):
    mu = jnp.mean(x, axis=-1, keepdims=True)
    xc = x - mu
    var = jnp.mean(xc * xc, axis=-1, keepdims=True)
    h = xc * lax.rsqrt(var + 1e-5) * lns + lnb
    logits = jnp.dot(h, wr, preferred_element_type=jnp.float32)
    logits = logits - jnp.max(logits, axis=-1, keepdims=True)
    ex = jnp.exp(logits)
    probs = ex / jnp.sum(ex, axis=-1, keepdims=True)
    m1 = jnp.max(probs, axis=-1, keepdims=True)
    eio = lax.broadcasted_iota(jnp.int32, probs.shape, 1)
    i1 = jnp.min(jnp.where(probs >= m1, eio, E), axis=-1, keepdims=True)
    mask1 = eio == i1
    probs_wo = jnp.where(mask1, -1.0, probs)
    m2 = jnp.max(probs_wo, axis=-1, keepdims=True)
    i2 = jnp.min(jnp.where(probs_wo >= m2, eio, E), axis=-1, keepdims=True)
    mask2 = eio == i2
    return h, m1, m2, mask1, mask2


def _route_kernel(x_ref, wr_ref, lns_ref, lnb_ref,
                  h_ref, pos_ref, w_ref, meta_ref):
    x = x_ref[...]
    h, m1, m2, mask1, mask2 = _ln_router(x, wr_ref[...], lns_ref[...],
                                         lnb_ref[...])
    h_ref[...] = _pack2(h)
    denom = m1 + m2
    w_ref[...] = jnp.concatenate([m1 / denom, m2 / denom], axis=1)

    m01 = (mask1 | mask2).astype(jnp.float32)

    CH = 256
    rio = lax.broadcasted_iota(jnp.int32, (CH, CH), 0)
    cio = lax.broadcasted_iota(jnp.int32, (CH, CH), 1)
    ltri = (cio < rio).astype(jnp.bfloat16)
    carry = jnp.zeros((1, E), jnp.float32)
    rank_chunks = []
    for c in range(T // CH):
        mb = m01[c * CH:(c + 1) * CH, :]
        ranks_c = jnp.dot(ltri, mb.astype(jnp.bfloat16),
                          preferred_element_type=jnp.float32) + carry
        rank_chunks.append(ranks_c)
        carry = carry + jnp.sum(mb, axis=0, keepdims=True)
    ranks = jnp.concatenate(rank_chunks, axis=0)
    counts = carry

    tiles = jnp.floor((counts + (TM - 1)) * (1.0 / TM))
    uio_r = lax.broadcasted_iota(jnp.int32, (E, E), 0)
    uio_c = lax.broadcasted_iota(jnp.int32, (E, E), 1)
    utri = (uio_r < uio_c).astype(jnp.bfloat16)
    start_tile = jnp.dot(tiles.astype(jnp.bfloat16), utri,
                         preferred_element_type=jnp.float32)
    n_tiles = jnp.sum(tiles, axis=-1, keepdims=True)
    start_row = start_tile * TM

    sel1 = jnp.sum(jnp.where(mask1, start_row + ranks, 0.0), axis=-1,
                   keepdims=True)
    sel2 = jnp.sum(jnp.where(mask2, start_row + ranks, 0.0), axis=-1,
                   keepdims=True)
    pos_ref[...] = jnp.concatenate([sel1, sel2], axis=1).astype(jnp.int32)

    end_tile = start_tile + tiles
    live = (tiles > 0.0).astype(jnp.float32)
    mio = lax.broadcasted_iota(jnp.int32, (32, E), 0).astype(jnp.float32)
    texp = jnp.sum((mio >= end_tile).astype(jnp.float32), axis=-1,
                   keepdims=True)
    e_last = jnp.sum((end_tile <= n_tiles - 1.0).astype(jnp.float32),
                     axis=-1, keepdims=True)
    texp = jnp.minimum(texp, e_last)
    eio = lax.broadcasted_iota(jnp.int32, (32, E), 1).astype(jnp.float32)
    eq = (eio == texp).astype(jnp.float32)
    f_cur = jnp.sum(eq * start_tile, axis=-1, keepdims=True)
    f_next = jnp.sum(eq * end_tile, axis=-1, keepdims=True)
    ordv = jnp.sum((mio >= end_tile).astype(jnp.float32) * live, axis=-1,
                   keepdims=True)
    sio = lax.broadcasted_iota(jnp.int32, (32, 1), 0)
    texp = jnp.where(sio == MT, n_tiles, texp)
    meta_col = jnp.concatenate([texp, f_cur, f_next, ordv],
                               axis=0).astype(jnp.int32)
    meta_ref[...] = jnp.broadcast_to(meta_col, (128, 128))


def _route(tokens, Wr, lns, lnb):
    return pl.pallas_call(
        _route_kernel,
        out_shape=(
            jax.ShapeDtypeStruct((T, DH), jnp.int32),
            jax.ShapeDtypeStruct((T, K), jnp.int32),
            jax.ShapeDtypeStruct((T, K), jnp.float32),
            jax.ShapeDtypeStruct((128, 128), jnp.int32),
        ),
    )(tokens, Wr, lns, lnb)


def _sc_scatter(h_packed, pos_flat):
    mesh = plsc.VectorSubcoreMesh(core_axis_name="c", subcore_axis_name="s")

    @functools.partial(
        pl.kernel, mesh=mesh,
        out_type=jax.ShapeDtypeStruct((A_PAD, DH), jnp.int32),
        scratch_types=[
            pltpu.VMEM((_BPW,), jnp.int32),
            pltpu.VMEM((_BPW, DH), jnp.int32),
            pltpu.SemaphoreType.DMA,
        ],
    )
    def k(h_hbm, idx_hbm, xg_hbm, idx_v, rows_v, sem):
        wid = lax.axis_index("s") * 2 + lax.axis_index("c")
        base = wid * _BPW
        pltpu.sync_copy(idx_hbm.at[pl.ds(base, _BPW)], idx_v)
        pltpu.sync_copy(h_hbm.at[pl.ds(lax.rem(base, T), _BPW)], rows_v)
        pltpu.async_copy(rows_v, xg_hbm.at[idx_v], sem).wait()

    return k(h_packed, pos_flat)


def _sc_gather(y_g, pos_flat):
    mesh = plsc.VectorSubcoreMesh(core_axis_name="c", subcore_axis_name="s")

    @functools.partial(
        pl.kernel, mesh=mesh,
        out_type=jax.ShapeDtypeStruct((A, DH), jnp.int32),
        scratch_types=[
            pltpu.VMEM((_BPW,), jnp.int32),
            pltpu.VMEM((_BPW, DH), jnp.int32),
            pltpu.SemaphoreType.DMA,
        ],
    )
    def k(yg_hbm, idx_hbm, yp_hbm, idx_v, rows_v, sem):
        wid = lax.axis_index("s") * 2 + lax.axis_index("c")
        base = wid * _BPW
        pltpu.sync_copy(idx_hbm.at[pl.ds(base, _BPW)], idx_v)
        pltpu.async_copy(yg_hbm.at[idx_v], rows_v, sem).wait()
        pltpu.sync_copy(rows_v, yp_hbm.at[pl.ds(base, _BPW)])

    return k(y_g, pos_flat)


_LOOKAHEAD = 3
_WCHUNK = 4


def _w_copies(w1_hbm, w2_hbm, e, w1b, w2b, sem1, sem2, slot):
    half = F // _WCHUNK
    cps = []
    for c in range(_WCHUNK):
        cps.append(pltpu.make_async_copy(
            w1_hbm.at[e, :, pl.ds(c * half, half)],
            w1b.at[slot, :, pl.ds(c * half, half)],
            sem1.at[slot, c]))
        cps.append(pltpu.make_async_copy(
            w2_hbm.at[e, pl.ds(c * half, half), :],
            w2b.at[slot, pl.ds(c * half, half), :],
            sem2.at[slot, c]))
    return cps


def _ffn_kernel(s_ref, x_ref, w1_hbm, w2_hbm, y_ref,
                w1b, w2b, sem1, sem2):
    i = pl.program_id(0)
    n = s_ref[MT]

    @pl.when(i < n)
    def _():
        e_cur = s_ref[i]
        f_cur = s_ref[32 + i]
        f_nxt = s_ref[64 + i]
        ordv = s_ref[96 + i]
        slot = lax.rem(ordv, 2)
        nslot = 1 - slot

        @pl.when(i == 0)
        def _():
            for cp in _w_copies(w1_hbm, w2_hbm, e_cur, w1b, w2b,
                                sem1, sem2, slot):
                cp.start()

        issue_at = jnp.maximum(f_nxt - _LOOKAHEAD, f_cur)

        @pl.when((i == issue_at) & (f_nxt < n))
        def _():
            e_nxt = s_ref[jnp.minimum(f_nxt, MT - 1)]
            for cp in _w_copies(w1_hbm, w2_hbm, e_nxt, w1b, w2b,
                                sem1, sem2, nslot):
                cp.start()

        @pl.when(i == f_cur)
        def _():
            for cp in _w_copies(w1_hbm, w2_hbm, e_cur, w1b, w2b,
                                sem1, sem2, slot):
                cp.wait()

        xb = _unpack2(x_ref[...]).astype(jnp.bfloat16)
        w1 = w1b[slot].astype(jnp.bfloat16)
        mid = jnp.dot(xb, w1, preferred_element_type=jnp.float32)
        mid = mid * jax.nn.sigmoid(mid)
        y = jnp.dot(mid.astype(jnp.bfloat16), w2b[slot].astype(jnp.bfloat16),
                    preferred_element_type=jnp.float32)
        y_ref[...] = _pack2(y)


def _ffn(meta, x_g, W1, W2):
    grid_spec = pltpu.PrefetchScalarGridSpec(
        num_scalar_prefetch=1,
        grid=(MT,),
        in_specs=[
            pl.BlockSpec((TM, DH),
                         lambda i, s: (jnp.minimum(i, s[MT] - 1), 0)),
            pl.BlockSpec(memory_space=pltpu.MemorySpace.HBM),
            pl.BlockSpec(memory_space=pltpu.MemorySpace.HBM),
        ],
        out_specs=pl.BlockSpec((TM, DH),
                               lambda i, s: (jnp.minimum(i, s[MT] - 1), 0)),
        scratch_shapes=[
            pltpu.VMEM((2, D, F), jnp.float32),
            pltpu.VMEM((2, F, D), jnp.float32),
            pltpu.SemaphoreType.DMA((2, _WCHUNK)),
            pltpu.SemaphoreType.DMA((2, _WCHUNK)),
        ],
    )
    return pl.pallas_call(
        _ffn_kernel,
        grid_spec=grid_spec,
        out_shape=jax.ShapeDtypeStruct((A_PAD, DH), jnp.int32),
    )(meta, x_g, W1, W2)


def _combine_kernel(x_ref, w_ref, ya_ref, yb_ref, out_ref):
    w = w_ref[...]
    ya = _unpack2(ya_ref[...])
    yb = _unpack2(yb_ref[...])
    out_ref[...] = x_ref[...] + w[:, 0:1] * ya + w[:, 1:2] * yb


def _combine(tokens, wpair, y_pair):
    return pl.pallas_call(
        _combine_kernel,
        grid=(N_T,),
        in_specs=[
            pl.BlockSpec((TM, D), lambda t: (t, 0)),
            pl.BlockSpec((TM, K), lambda t: (t, 0)),
            pl.BlockSpec((TM, DH), lambda t: (t, 0)),
            pl.BlockSpec((TM, DH), lambda t: (t + N_T, 0)),
        ],
        out_specs=pl.BlockSpec((TM, D), lambda t: (t, 0)),
        out_shape=jax.ShapeDtypeStruct((T, D), jnp.float32),
    )(tokens, wpair, y_pair, y_pair)


@jax.jit
def kernel(hidden_states, Wr, W1, W2, ln_scale, ln_bias):
    b, s, d = hidden_states.shape
    tokens = hidden_states.reshape(T, D)
    lns = ln_scale.reshape(1, D)
    lnb = ln_bias.reshape(1, D)

    h_packed, posw, wpair, meta2d = _route(tokens, Wr, lns, lnb)
    pos_flat = posw.T.reshape(A)
    meta = meta2d[:, 0]

    x_g = _sc_scatter(h_packed, pos_flat)
    y_g = _ffn(meta, x_g, W1, W2)
    y_pair = _sc_gather(y_g, pos_flat)
    out = _combine(tokens, wpair, y_pair)
    return out.reshape(b, s, d)
